# Initial kernel scaffold; baseline (speedup 1.0000x reference)
import functools

import jax
import jax.numpy as jnp
from jax.experimental import pallas as pl
from jax.experimental.pallas import tpu as pltpu

_K = 8


def _fused_body(xyz_ref, x_ref, w1_ref, w2_ref, t_ref, out_ref, z_scr, *, rows, k):
    ti = pl.program_id(1)
    n = x_ref.shape[1]
    r0 = ti * rows

    @pl.when(ti == 0)
    def _():
        z_scr[...] = jnp.dot(x_ref[0], w1_ref[...],
                             preferred_element_type=jnp.float32)

    xt = x_ref[0, pl.ds(r0, rows), :]
    base = jnp.dot(xt, w2_ref[...],
                   preferred_element_type=jnp.float32) + t_ref[...]

    xyzb = xyz_ref[0]
    xx = (xyzb[0:1, :] * xyzb[0:1, :]
          + xyzb[1:2, :] * xyzb[1:2, :]
          + xyzb[2:3, :] * xyzb[2:3, :])
    xi0 = xyzb[0:1, pl.ds(r0, rows)]
    xi1 = xyzb[1:2, pl.ds(r0, rows)]
    xi2 = xyzb[2:3, pl.ds(r0, rows)]
    inner = -2.0 * (jnp.transpose(xi0) * xyzb[0:1, :]
                    + jnp.transpose(xi1) * xyzb[1:2, :]
                    + jnp.transpose(xi2) * xyzb[2:3, :])
    xxi = (jnp.transpose(xi0) * jnp.transpose(xi0)
           + jnp.transpose(xi1) * jnp.transpose(xi1)
           + jnp.transpose(xi2) * jnp.transpose(xi2))
    d = (-xx - inner) - xxi

    iota = jax.lax.broadcasted_iota(jnp.int32, (rows, n), 1)
    acc = jnp.full((rows, out_ref.shape[2]), -jnp.inf, dtype=jnp.float32)
    for _ in range(k):
        m = jnp.max(d, axis=1, keepdims=True)
        sel = d == m
        idxv = jnp.min(jnp.where(sel, iota, n), axis=1, keepdims=True)
        oh = iota == idxv
        zg = jnp.dot(oh.astype(jnp.float32), z_scr[...],
                     preferred_element_type=jnp.float32)
        y = zg + base
        y = jnp.maximum(y, 0.2 * y)
        acc = jnp.maximum(acc, y)
        d = jnp.where(oh, -jnp.inf, d)
    out_ref[0] = acc


def kernel(inputs, xyz, W, gamma, beta, bn_mean, bn_var):
    b, n, c = inputs.shape
    out_dim = W.shape[0]
    rows = min(256, n)

    s = gamma * jax.lax.rsqrt(bn_var + 1e-3)
    t = (beta - bn_mean * s).reshape(1, out_dim)
    w1 = (W[:, :c] * s[:, None]).T
    w2 = ((W[:, c:] - W[:, :c]) * s[:, None]).T
    xyz = xyz.reshape(b, 3, n)

    out = pl.pallas_call(
        functools.partial(_fused_body, rows=rows, k=_K),
        grid=(b, n // rows),
        in_specs=[
            pl.BlockSpec((1, 3, n), lambda bi, ti: (bi, 0, 0)),
            pl.BlockSpec((1, n, c), lambda bi, ti: (bi, 0, 0)),
            pl.BlockSpec((c, out_dim), lambda bi, ti: (0, 0)),
            pl.BlockSpec((c, out_dim), lambda bi, ti: (0, 0)),
            pl.BlockSpec((1, out_dim), lambda bi, ti: (0, 0)),
        ],
        out_specs=pl.BlockSpec((1, rows, out_dim), lambda bi, ti: (bi, ti, 0)),
        out_shape=jax.ShapeDtypeStruct((b, n, out_dim), jnp.float32),
        scratch_shapes=[pltpu.VMEM((n, out_dim), jnp.float32)],
    )(xyz, inputs, w1, w2, t)
    return out


# speedup vs baseline: 20.8009x; 20.8009x over previous
import functools

import jax
import jax.numpy as jnp
from jax.experimental import pallas as pl
from jax.experimental.pallas import tpu as pltpu

_K = 8


def _fused_body(xyz_ref, xyzt_ref, xyzb_ref, xyztb_ref, x_ref, w1_ref, w2_ref,
                t_ref, out_ref, z_scr, *, rows, k):
    ti = pl.program_id(1)
    n = x_ref.shape[1]
    r0 = ti * rows

    @pl.when(ti == 0)
    def _():
        z_scr[...] = jnp.dot(x_ref[0], w1_ref[...],
                             preferred_element_type=jnp.float32)

    xt = x_ref[0, pl.ds(r0, rows), :]
    base = jnp.dot(xt, w2_ref[...],
                   preferred_element_type=jnp.float32) + t_ref[...]

    xj0 = xyz_ref[0, 0:1, :]
    xj1 = xyz_ref[0, 1:2, :]
    xj2 = xyz_ref[0, 2:3, :]
    xx = xj0 * xj0 + xj1 * xj1 + xj2 * xj2
    xit = xyzt_ref[0, pl.ds(r0, rows), :]
    xi0 = xit[:, 0:1]
    xi1 = xit[:, 1:2]
    xi2 = xit[:, 2:3]
    xitb = xyztb_ref[0, pl.ds(r0, rows), :]
    inner = -2.0 * jnp.dot(xitb, xyzb_ref[0],
                           preferred_element_type=jnp.float32)
    xxi = xi0 * xi0 + xi1 * xi1 + xi2 * xi2
    d = (-xx - inner) - xxi

    iota = jax.lax.broadcasted_iota(jnp.int32, (rows, n), 1)
    acc = jnp.full((rows, out_ref.shape[2]), -jnp.inf, dtype=jnp.float32)
    for _ in range(k):
        m = jnp.max(d, axis=1, keepdims=True)
        sel = d == m
        idxv = jnp.min(jnp.where(sel, iota, n), axis=1, keepdims=True)
        oh = iota == idxv
        zg = jnp.dot(oh.astype(jnp.float32), z_scr[...],
                     preferred_element_type=jnp.float32)
        y = zg + base
        y = jnp.maximum(y, 0.2 * y)
        acc = jnp.maximum(acc, y)
        d = jnp.where(oh, -jnp.inf, d)
    out_ref[0] = acc


def kernel(inputs, xyz, W, gamma, beta, bn_mean, bn_var):
    b, n, c = inputs.shape
    out_dim = W.shape[0]
    rows = min(256, n)

    s = gamma * jax.lax.rsqrt(bn_var + 1e-3)
    t = (beta - bn_mean * s).reshape(1, out_dim)
    w1 = (W[:, :c] * s[:, None]).T
    w2 = ((W[:, c:] - W[:, :c]) * s[:, None]).T
    xyz = xyz.reshape(b, 3, n)
    xyzt = jnp.transpose(xyz, (0, 2, 1))
    xyzb = xyz.astype(jnp.bfloat16)
    xyztb = xyzt.astype(jnp.bfloat16)

    out = pl.pallas_call(
        functools.partial(_fused_body, rows=rows, k=_K),
        grid=(b, n // rows),
        in_specs=[
            pl.BlockSpec((1, 3, n), lambda bi, ti: (bi, 0, 0)),
            pl.BlockSpec((1, n, 3), lambda bi, ti: (bi, 0, 0)),
            pl.BlockSpec((1, 3, n), lambda bi, ti: (bi, 0, 0)),
            pl.BlockSpec((1, n, 3), lambda bi, ti: (bi, 0, 0)),
            pl.BlockSpec((1, n, c), lambda bi, ti: (bi, 0, 0)),
            pl.BlockSpec((c, out_dim), lambda bi, ti: (0, 0)),
            pl.BlockSpec((c, out_dim), lambda bi, ti: (0, 0)),
            pl.BlockSpec((1, out_dim), lambda bi, ti: (0, 0)),
        ],
        out_specs=pl.BlockSpec((1, rows, out_dim), lambda bi, ti: (bi, ti, 0)),
        out_shape=jax.ShapeDtypeStruct((b, n, out_dim), jnp.float32),
        scratch_shapes=[pltpu.VMEM((n, out_dim), jnp.float32)],
    )(xyz, xyzt, xyzb, xyztb, inputs, w1, w2, t)
    return out
